# baseline (speedup 1.0000x reference)
import functools

import numpy as np

import jax
import jax.numpy as jnp
from jax.experimental import pallas as pl
from jax.experimental.pallas import tpu as pltpu
from jax.sharding import Mesh, PartitionSpec as P

NPC = 1024
K = 16

_HI = jax.lax.Precision.HIGHEST
_LO = jax.lax.Precision.DEFAULT


def _cloud_kernel(pos_ref, x_ref,
                  W10_ref, b10_ref, g10_ref, be10_ref,
                  W11_ref, b11_ref, g11_ref, be11_ref,
                  W12_ref, b12_ref, g12_ref, be12_ref,
                  W2_ref, b2_ref, g2_ref, be2_ref,
                  Wlin_ref, blin_ref,
                  out_ref, d_ref, w_ref):
    f32 = jnp.float32
    x0 = jnp.concatenate([pos_ref[...], x_ref[...]], axis=1)

    gram = jax.lax.dot_general(x0, x0, (((1,), (1,)), ((), ())),
                               precision=_HI)
    sq = jnp.sum(x0 * x0, axis=1, keepdims=True)
    d_ref[...] = sq - 2.0 * gram

    x08 = jnp.concatenate([x0, -x0], axis=1)
    A1 = jnp.dot(x08, W10_ref[...], precision=_HI) + b10_ref[...]
    z4 = jnp.zeros_like(x0)
    B1 = jnp.dot(jnp.concatenate([z4, x0], axis=1), W10_ref[...], precision=_HI)
    B1b = B1.astype(jnp.bfloat16)
    W11b = W11_ref[...].astype(jnp.bfloat16)
    W12b = W12_ref[...].astype(jnp.bfloat16)

    m0 = jnp.min(d_ref[...], axis=0, keepdims=True)

    def conv1_step(k, carry):
        m_cur, x1 = carry
        keys = d_ref[...]
        w_ref[pl.ds(k, 1)] = m_cur
        m_next = jnp.min(jnp.where(keys > m_cur, keys, jnp.inf), axis=0,
                         keepdims=True)
        oh = (keys == m_cur).astype(jnp.bfloat16)
        b1j = jax.lax.dot_general(oh, B1b, (((0,), (0,)), ((), ())),
                                  precision=_LO,
                                  preferred_element_type=f32)
        t = jnp.maximum(A1 + b1j, 0.0) * g10_ref[...] + be10_ref[...]
        t = (jnp.maximum(jnp.dot(t.astype(jnp.bfloat16), W11b, precision=_LO,
                                 preferred_element_type=f32) + b11_ref[...],
                         0.0) * g11_ref[...] + be11_ref[...])
        t = (jnp.maximum(jnp.dot(t.astype(jnp.bfloat16), W12b, precision=_LO,
                                 preferred_element_type=f32) + b12_ref[...],
                         0.0) * g12_ref[...] + be12_ref[...])
        return m_next, jnp.maximum(x1, t)

    _, x1 = jax.lax.fori_loop(0, K, conv1_step,
                              (m0, jnp.full((NPC, 64), -jnp.inf, f32)))

    W2a = W2_ref[0:64, :]
    W2b = W2_ref[64:128, :]
    R = jnp.dot(x1, W2a - W2b, precision=_HI) + b2_ref[...]
    x1b = x1.astype(jnp.bfloat16)
    W2bb = W2b.astype(jnp.bfloat16)

    def conv2_step(k, x2):
        m_k = w_ref[pl.ds(k, 1)]
        sel = (d_ref[...] == m_k).astype(jnp.bfloat16)
        x1j = jax.lax.dot_general(sel, x1b, (((0,), (0,)), ((), ())),
                                  precision=_LO,
                                  preferred_element_type=f32)
        q = jnp.dot(x1j.astype(jnp.bfloat16), W2bb, precision=_LO,
                    preferred_element_type=f32)
        t = jnp.maximum(R + q, 0.0) * g2_ref[...] + be2_ref[...]
        return jnp.maximum(x2, t)

    x2 = jax.lax.fori_loop(0, K, conv2_step,
                           jnp.full((NPC, 128), -jnp.inf, f32))

    out1 = (jnp.dot(jnp.concatenate([x1, x2], axis=1).astype(jnp.bfloat16),
                    Wlin_ref[...].astype(jnp.bfloat16), precision=_LO,
                    preferred_element_type=f32)
            + blin_ref[...])
    out_ref[0] = jnp.max(out1, axis=0, keepdims=True)


def _head_kernel(p_ref, Wh0_ref, bh0_ref, Wh1_ref, bh1_ref, Wh2_ref, bh2_ref,
                 out_ref):
    h = jnp.maximum(jnp.dot(p_ref[...], Wh0_ref[...], precision=_HI)
                    + bh0_ref[...], 0.0)
    h = jnp.maximum(jnp.dot(h, Wh1_ref[...], precision=_HI) + bh1_ref[...], 0.0)
    l = jnp.dot(h, Wh2_ref[...], precision=_HI) + bh2_ref[...]
    m = jnp.max(l, axis=1, keepdims=True)
    out_ref[...] = (l - m) - jnp.log(jnp.sum(jnp.exp(l - m), axis=1,
                                             keepdims=True))


def _full(shape):
    return pl.BlockSpec(shape, lambda *a: tuple(0 for _ in shape))


def _cloud_call(pos, x, W1_0, b1_0, g1_0, be1_0, W1_1, b1_1, g1_1, be1_1,
                W1_2, b1_2, g1_2, be1_2, W2, b2, g2, be2, Wlin, blin,
                interpret=False):
    nb = pos.shape[0] // NPC
    pooled = pl.pallas_call(
        _cloud_kernel,
        grid=(nb,),
        in_specs=[
            pl.BlockSpec((NPC, 3), lambda c: (c, 0)),
            pl.BlockSpec((NPC, 1), lambda c: (c, 0)),
            _full(W1_0.shape), _full(b1_0.shape), _full(g1_0.shape),
            _full(be1_0.shape),
            _full(W1_1.shape), _full(b1_1.shape), _full(g1_1.shape),
            _full(be1_1.shape),
            _full(W1_2.shape), _full(b1_2.shape), _full(g1_2.shape),
            _full(be1_2.shape),
            _full(W2.shape), _full(b2.shape), _full(g2.shape), _full(be2.shape),
            _full(Wlin.shape), _full(blin.shape),
        ],
        out_specs=pl.BlockSpec((1, 1, 1024), lambda c: (c, 0, 0)),
        out_shape=jax.ShapeDtypeStruct((nb, 1, 1024), jnp.float32),
        scratch_shapes=[pltpu.VMEM((NPC, NPC), jnp.float32),
                        pltpu.VMEM((K, NPC), jnp.float32)],
        interpret=interpret,
    )(pos, x, W1_0, b1_0, g1_0, be1_0, W1_1, b1_1, g1_1, be1_1,
      W1_2, b1_2, g1_2, be1_2, W2, b2, g2, be2, Wlin, blin)
    return pooled.reshape(nb, 1024)


def _head_call(pooled, Wh0, bh0, Wh1, bh1, Wh2, bh2, interpret=False):
    nb = pooled.shape[0]
    logp = pl.pallas_call(
        _head_kernel,
        in_specs=[_full(pooled.shape), _full(Wh0.shape), _full(bh0.shape),
                  _full(Wh1.shape), _full(bh1.shape), _full(Wh2.shape),
                  _full(bh2.shape)],
        out_specs=_full((nb, Wh2.shape[1])),
        out_shape=jax.ShapeDtypeStruct((nb, Wh2.shape[1]), jnp.float32),
        interpret=interpret,
    )(pooled, Wh0, bh0, Wh1, bh1, Wh2, bh2)
    return logp


def _forward(pos, x, batch, *ws, interpret=False):
    del batch
    cloud_ws, head_ws = ws[:18], ws[18:]
    nb = pos.shape[0] // NPC

    devs = jax.devices()
    nd = 2 if (len(devs) >= 2 and nb % 2 == 0) else 1
    if nd == 1:
        pooled = _cloud_call(pos, x, *cloud_ws, interpret=interpret)
        return _head_call(pooled, *head_ws, interpret=interpret)

    mesh = Mesh(np.asarray(devs[:nd]), ("d",))

    def sharded(pos_l, x_l, *ws_l):
        pooled_l = _cloud_call(pos_l, x_l, *ws_l[:18], interpret=interpret)
        pooled = jax.lax.all_gather(pooled_l, "d", axis=0, tiled=True)
        return _head_call(pooled, *ws_l[18:], interpret=interpret)

    f = jax.shard_map(sharded, mesh=mesh,
                      in_specs=(P("d"), P("d")) + (P(),) * len(ws),
                      out_specs=P(), check_vma=False)
    return f(pos, x, *ws)


def kernel(pos, x, batch, W1_0, b1_0, g1_0, be1_0, W1_1, b1_1, g1_1, be1_1,
           W1_2, b1_2, g1_2, be1_2, W2, b2, g2, be2, Wlin, blin,
           Wh0, bh0, Wh1, bh1, Wh2, bh2):
    return _forward(pos, x, batch, W1_0, b1_0, g1_0, be1_0, W1_1, b1_1, g1_1,
                    be1_1, W1_2, b1_2, g1_2, be1_2, W2, b2, g2, be2, Wlin,
                    blin, Wh0, bh0, Wh1, bh1, Wh2, bh2)


# speedup vs baseline: 1.0166x; 1.0166x over previous
import jax
import jax.numpy as jnp
from jax.experimental import pallas as pl
from jax.experimental.pallas import tpu as pltpu

NPC = 1024
K = 16

_HI = jax.lax.Precision.HIGHEST
_LO = jax.lax.Precision.DEFAULT


def _cloud_kernel(pos_ref, x_ref,
                  W10_ref, b10_ref, g10_ref, be10_ref,
                  W11_ref, b11_ref, g11_ref, be11_ref,
                  W12_ref, b12_ref, g12_ref, be12_ref,
                  W2_ref, b2_ref, g2_ref, be2_ref,
                  Wlin_ref, blin_ref,
                  out_ref, d_ref, w_ref):
    f32 = jnp.float32
    x0 = jnp.concatenate([pos_ref[...], x_ref[...]], axis=1)

    gram = jax.lax.dot_general(x0, x0, (((1,), (1,)), ((), ())),
                               precision=_HI)
    sq = jnp.sum(x0 * x0, axis=1, keepdims=True)
    d_ref[...] = sq - 2.0 * gram

    x08 = jnp.concatenate([x0, -x0], axis=1)
    A1 = jnp.dot(x08, W10_ref[...], precision=_HI) + b10_ref[...]
    z4 = jnp.zeros_like(x0)
    B1 = jnp.dot(jnp.concatenate([z4, x0], axis=1), W10_ref[...], precision=_HI)
    B1b = B1.astype(jnp.bfloat16)
    W11b = W11_ref[...].astype(jnp.bfloat16)
    W12b = W12_ref[...].astype(jnp.bfloat16)

    m0 = jnp.min(d_ref[...], axis=0, keepdims=True)

    def conv1_step(k, carry):
        m_cur, x1 = carry
        keys = d_ref[...]
        w_ref[pl.ds(k, 1)] = m_cur
        m_next = jnp.min(jnp.where(keys > m_cur, keys, jnp.inf), axis=0,
                         keepdims=True)
        oh = (keys == m_cur).astype(jnp.bfloat16)
        b1j = jax.lax.dot_general(oh, B1b, (((0,), (0,)), ((), ())),
                                  precision=_LO,
                                  preferred_element_type=f32)
        t = jnp.maximum(A1 + b1j, 0.0) * g10_ref[...] + be10_ref[...]
        t = (jnp.maximum(jnp.dot(t.astype(jnp.bfloat16), W11b, precision=_LO,
                                 preferred_element_type=f32) + b11_ref[...],
                         0.0) * g11_ref[...] + be11_ref[...])
        t = (jnp.maximum(jnp.dot(t.astype(jnp.bfloat16), W12b, precision=_LO,
                                 preferred_element_type=f32) + b12_ref[...],
                         0.0) * g12_ref[...] + be12_ref[...])
        return m_next, jnp.maximum(x1, t)

    _, x1 = jax.lax.fori_loop(0, K, conv1_step,
                              (m0, jnp.full((NPC, 64), -jnp.inf, f32)))

    W2a = W2_ref[0:64, :]
    W2b = W2_ref[64:128, :]
    R = jnp.dot(x1, W2a - W2b, precision=_HI) + b2_ref[...]
    x1b = x1.astype(jnp.bfloat16)
    W2bb = W2b.astype(jnp.bfloat16)

    def conv2_step(k, x2):
        m_k = w_ref[pl.ds(k, 1)]
        sel = (d_ref[...] == m_k).astype(jnp.bfloat16)
        x1j = jax.lax.dot_general(sel, x1b, (((0,), (0,)), ((), ())),
                                  precision=_LO,
                                  preferred_element_type=f32)
        q = jnp.dot(x1j.astype(jnp.bfloat16), W2bb, precision=_LO,
                    preferred_element_type=f32)
        t = jnp.maximum(R + q, 0.0) * g2_ref[...] + be2_ref[...]
        return jnp.maximum(x2, t)

    x2 = jax.lax.fori_loop(0, K, conv2_step,
                           jnp.full((NPC, 128), -jnp.inf, f32))

    out1 = (jnp.dot(jnp.concatenate([x1, x2], axis=1).astype(jnp.bfloat16),
                    Wlin_ref[...].astype(jnp.bfloat16), precision=_LO,
                    preferred_element_type=f32)
            + blin_ref[...])
    out_ref[0] = jnp.max(out1, axis=0, keepdims=True)


def _head_kernel(p_ref, Wh0_ref, bh0_ref, Wh1_ref, bh1_ref, Wh2_ref, bh2_ref,
                 out_ref):
    h = jnp.maximum(jnp.dot(p_ref[...], Wh0_ref[...], precision=_HI)
                    + bh0_ref[...], 0.0)
    h = jnp.maximum(jnp.dot(h, Wh1_ref[...], precision=_HI) + bh1_ref[...], 0.0)
    l = jnp.dot(h, Wh2_ref[...], precision=_HI) + bh2_ref[...]
    m = jnp.max(l, axis=1, keepdims=True)
    out_ref[...] = (l - m) - jnp.log(jnp.sum(jnp.exp(l - m), axis=1,
                                             keepdims=True))


def _full(shape):
    return pl.BlockSpec(shape, lambda *a: tuple(0 for _ in shape))


def _cloud_call(pos, x, W1_0, b1_0, g1_0, be1_0, W1_1, b1_1, g1_1, be1_1,
                W1_2, b1_2, g1_2, be1_2, W2, b2, g2, be2, Wlin, blin,
                interpret=False):
    nb = pos.shape[0] // NPC
    pooled = pl.pallas_call(
        _cloud_kernel,
        grid=(nb,),
        in_specs=[
            pl.BlockSpec((NPC, 3), lambda c: (c, 0)),
            pl.BlockSpec((NPC, 1), lambda c: (c, 0)),
            _full(W1_0.shape), _full(b1_0.shape), _full(g1_0.shape),
            _full(be1_0.shape),
            _full(W1_1.shape), _full(b1_1.shape), _full(g1_1.shape),
            _full(be1_1.shape),
            _full(W1_2.shape), _full(b1_2.shape), _full(g1_2.shape),
            _full(be1_2.shape),
            _full(W2.shape), _full(b2.shape), _full(g2.shape), _full(be2.shape),
            _full(Wlin.shape), _full(blin.shape),
        ],
        out_specs=pl.BlockSpec((1, 1, 1024), lambda c: (c, 0, 0)),
        out_shape=jax.ShapeDtypeStruct((nb, 1, 1024), jnp.float32),
        scratch_shapes=[pltpu.VMEM((NPC, NPC), jnp.float32),
                        pltpu.VMEM((K, NPC), jnp.float32)],
        interpret=interpret,
    )(pos, x, W1_0, b1_0, g1_0, be1_0, W1_1, b1_1, g1_1, be1_1,
      W1_2, b1_2, g1_2, be1_2, W2, b2, g2, be2, Wlin, blin)
    return pooled.reshape(nb, 1024)


def _head_call(pooled, Wh0, bh0, Wh1, bh1, Wh2, bh2, interpret=False):
    nb = pooled.shape[0]
    logp = pl.pallas_call(
        _head_kernel,
        in_specs=[_full(pooled.shape), _full(Wh0.shape), _full(bh0.shape),
                  _full(Wh1.shape), _full(bh1.shape), _full(Wh2.shape),
                  _full(bh2.shape)],
        out_specs=_full((nb, Wh2.shape[1])),
        out_shape=jax.ShapeDtypeStruct((nb, Wh2.shape[1]), jnp.float32),
        interpret=interpret,
    )(pooled, Wh0, bh0, Wh1, bh1, Wh2, bh2)
    return logp


def _forward(pos, x, batch, *ws, interpret=False):
    del batch
    cloud_ws, head_ws = ws[:18], ws[18:]
    pooled = _cloud_call(pos, x, *cloud_ws, interpret=interpret)
    return _head_call(pooled, *head_ws, interpret=interpret)


def kernel(pos, x, batch, W1_0, b1_0, g1_0, be1_0, W1_1, b1_1, g1_1, be1_1,
           W1_2, b1_2, g1_2, be1_2, W2, b2, g2, be2, Wlin, blin,
           Wh0, bh0, Wh1, bh1, Wh2, bh2):
    return _forward(pos, x, batch, W1_0, b1_0, g1_0, be1_0, W1_1, b1_1, g1_1,
                    be1_1, W1_2, b1_2, g1_2, be1_2, W2, b2, g2, be2, Wlin,
                    blin, Wh0, bh0, Wh1, bh1, Wh2, bh2)
